# baseline (speedup 1.0000x reference)
import functools

import jax
import jax.numpy as jnp
from jax import lax
from jax.experimental import pallas as pl
from jax.experimental.pallas import tpu as pltpu
from jax.experimental.pallas import tpu_sc as plsc

_NUM_WORKERS = 32
_NUM_CORES = 2


def _sc_body(x_hbm, out_hbm, stripe, n_rows, n_cols):
    wid = lax.axis_index("s") * _NUM_CORES + lax.axis_index("c")
    chunk = n_rows // _NUM_WORKERS
    base = wid * chunk
    pltpu.sync_copy(
        x_hbm.at[pl.ds(base, chunk), :], out_hbm.at[pl.ds(base, chunk), :]
    )

    lane = lax.iota(jnp.int32, 16)

    @pl.when(wid == 0)
    def _():
        pltpu.sync_copy(out_hbm.at[0, pl.ds(n_cols - 16, 16)], stripe)
        stripe[...] = jnp.where(lane == 14, jnp.float32(1.0), stripe[...])
        pltpu.sync_copy(stripe, out_hbm.at[0, pl.ds(n_cols - 16, 16)])

    @pl.when(wid == _NUM_WORKERS - 1)
    def _():
        pltpu.sync_copy(out_hbm.at[n_rows - 1, pl.ds(0, 16)], stripe)
        stripe[...] = jnp.where(lane == 1, jnp.float32(5.0), stripe[...])
        pltpu.sync_copy(stripe, out_hbm.at[n_rows - 1, pl.ds(0, 16)])


@jax.jit
def kernel(x):
    n_rows, n_cols = x.shape
    mesh = plsc.VectorSubcoreMesh(core_axis_name="c", subcore_axis_name="s")
    body = functools.partial(_sc_body, n_rows=n_rows, n_cols=n_cols)
    sc_call = pl.kernel(
        body,
        out_type=jax.ShapeDtypeStruct(x.shape, x.dtype),
        mesh=mesh,
        scratch_types=[pltpu.VMEM((16,), jnp.float32)],
    )
    return sc_call(x)


# speedup vs baseline: 38.8716x; 38.8716x over previous
import functools

import jax
import jax.numpy as jnp
from jax import lax
from jax.experimental import pallas as pl
from jax.experimental.pallas import tpu as pltpu
from jax.experimental.pallas import tpu_sc as plsc

_NUM_WORKERS = 32
_NUM_CORES = 2
_PIECE_ROWS = 8


def _sc_body(x_hbm, out_hbm, buf, stripe, in_sems, out_sems, n_rows, n_cols):
    wid = lax.axis_index("s") * _NUM_CORES + lax.axis_index("c")
    chunk = n_rows // _NUM_WORKERS
    base = wid * chunk
    num_pieces = chunk // _PIECE_ROWS

    def in_copy(g, slot):
        rows = pl.ds(base + g * _PIECE_ROWS, _PIECE_ROWS)
        return pltpu.make_async_copy(
            x_hbm.at[rows, :], buf.at[slot], in_sems.at[slot]
        )

    def out_copy(g, slot):
        rows = pl.ds(base + g * _PIECE_ROWS, _PIECE_ROWS)
        return pltpu.make_async_copy(
            buf.at[slot], out_hbm.at[rows, :], out_sems.at[slot]
        )

    in_copy(0, 0).start()
    in_copy(1, 1).start()

    def round_body(i, _):
        g0 = 2 * i
        for slot in (0, 1):
            g = g0 + slot
            in_copy(g, slot).wait()
            out_copy(g, slot).start()

        @pl.when(g0 + 2 < num_pieces)
        def _():
            for slot in (0, 1):
                g = g0 + slot
                out_copy(g, slot).wait()
                in_copy(g + 2, slot).start()

        return _

    lax.fori_loop(0, num_pieces // 2, round_body, None)
    for slot in (0, 1):
        out_copy(num_pieces - 2 + slot, slot).wait()

    lane = lax.iota(jnp.int32, 16)

    @pl.when(wid == 0)
    def _():
        pltpu.sync_copy(out_hbm.at[0, pl.ds(n_cols - 16, 16)], stripe)
        stripe[...] = jnp.where(lane == 14, jnp.float32(1.0), stripe[...])
        pltpu.sync_copy(stripe, out_hbm.at[0, pl.ds(n_cols - 16, 16)])

    @pl.when(wid == _NUM_WORKERS - 1)
    def _():
        pltpu.sync_copy(out_hbm.at[n_rows - 1, pl.ds(0, 16)], stripe)
        stripe[...] = jnp.where(lane == 1, jnp.float32(5.0), stripe[...])
        pltpu.sync_copy(stripe, out_hbm.at[n_rows - 1, pl.ds(0, 16)])


@jax.jit
def kernel(x):
    n_rows, n_cols = x.shape
    mesh = plsc.VectorSubcoreMesh(core_axis_name="c", subcore_axis_name="s")
    body = functools.partial(_sc_body, n_rows=n_rows, n_cols=n_cols)
    sc_call = pl.kernel(
        body,
        out_type=jax.ShapeDtypeStruct(x.shape, x.dtype),
        mesh=mesh,
        scratch_types=[
            pltpu.VMEM((2, _PIECE_ROWS, n_cols), jnp.float32),
            pltpu.VMEM((16,), jnp.float32),
            pltpu.SemaphoreType.DMA((2,)),
            pltpu.SemaphoreType.DMA((2,)),
        ],
    )
    return sc_call(x)


# speedup vs baseline: 49.0639x; 1.2622x over previous
import functools

import jax
import jax.numpy as jnp
from jax.experimental import pallas as pl
from jax.experimental.pallas import tpu as pltpu

_BLOCK_ROWS = 512


def _patch_tile(out_ref, rows, cols, row, col, value):
    tile = out_ref[rows, cols]
    r = jax.lax.broadcasted_iota(jnp.int32, tile.shape, 0)
    c = jax.lax.broadcasted_iota(jnp.int32, tile.shape, 1)
    mask = (r == row) & (c == col)
    out_ref[rows, cols] = jnp.where(mask, jnp.float32(value), tile)


def _copy_patch_body(in_ref, out_ref, *, n_cols, num_blocks, block_rows):
    out_ref[...] = in_ref[...]
    i = pl.program_id(0)

    @pl.when(i == 0)
    def _():
        _patch_tile(out_ref, pl.ds(0, 8), pl.ds(n_cols - 128, 128), 0, 126, 1.0)

    @pl.when(i == num_blocks - 1)
    def _():
        _patch_tile(out_ref, pl.ds(block_rows - 8, 8), pl.ds(0, 128), 7, 1, 5.0)


@jax.jit
def kernel(x):
    n_rows, n_cols = x.shape
    block_rows = _BLOCK_ROWS
    num_blocks = n_rows // block_rows
    body = functools.partial(
        _copy_patch_body,
        n_cols=n_cols,
        num_blocks=num_blocks,
        block_rows=block_rows,
    )
    return pl.pallas_call(
        body,
        grid=(num_blocks,),
        in_specs=[pl.BlockSpec((block_rows, n_cols), lambda i: (i, 0))],
        out_specs=pl.BlockSpec((block_rows, n_cols), lambda i: (i, 0)),
        out_shape=jax.ShapeDtypeStruct(x.shape, x.dtype),
        compiler_params=pltpu.CompilerParams(
            vmem_limit_bytes=100 * 1024 * 1024,
        ),
    )(x)
